# baseline (speedup 1.0000x reference)
import functools

import jax
import jax.numpy as jnp
from jax import lax
from jax.experimental import pallas as pl
from jax.experimental.pallas import tpu as pltpu
from jax.experimental.pallas import tpu_sc as plsc

NC = 2
NS = 16
L = 16
NROWS = 32
NCOLS = 4096
CHUNKS = NCOLS // L
GROUPS = NCOLS // 128

_mesh = plsc.VectorSubcoreMesh(
    core_axis_name="c", subcore_axis_name="s", num_cores=NC, num_subcores=NS
)


def _worker_id():
  return lax.axis_index("s") * NC + lax.axis_index("c")


@functools.partial(
    pl.kernel,
    compiler_params=pltpu.CompilerParams(needs_layout_passes=False),
    out_type=jax.ShapeDtypeStruct((NROWS, L), jnp.int32),
    mesh=_mesh,
    scratch_types=[
        pltpu.VMEM((NCOLS,), jnp.float32),
        pltpu.VMEM((L,), jnp.int32),
    ],
)
def _count_kernel(x_hbm, counts_hbm, xrow, cnt_v):
  w = _worker_id()
  pltpu.sync_copy(x_hbm.at[w], xrow)

  def step(c, acc):
    v = xrow[pl.ds(c * L, L)]
    return acc + jnp.where(v != 0.0, 1, 0).astype(jnp.int32)

  acc = lax.fori_loop(0, CHUNKS, step, jnp.zeros((L,), jnp.int32))
  total = jnp.sum(acc)
  cnt_v[...] = jnp.full((L,), total, jnp.int32)
  pltpu.sync_copy(cnt_v, counts_hbm.at[w])


@functools.partial(
    pl.kernel,
    compiler_params=pltpu.CompilerParams(needs_layout_passes=False),
    out_type=jax.ShapeDtypeStruct((NROWS * NCOLS,), jnp.int32),
    mesh=_mesh,
    scratch_types=[
        pltpu.VMEM((NCOLS,), jnp.float32),
        pltpu.VMEM((NROWS, L), jnp.int32),
        pltpu.VMEM((GROUPS, 128), jnp.int32),
        pltpu.VMEM((GROUPS, 128), jnp.int32),
        pltpu.SemaphoreType.DMA,
    ],
)
def _scatter_kernel(x_hbm, counts_hbm, codes_hbm, xrow, cnts_v, dest_b,
                    val_b, sem):
  w = _worker_id()
  pltpu.sync_copy(x_hbm.at[w], xrow)
  pltpu.sync_copy(counts_hbm, cnts_v)

  li = jnp.arange(L, dtype=jnp.int32)
  zeros = jnp.zeros((L,), jnp.int32)
  c0 = plsc.load_gather(cnts_v, [li, zeros])
  c1 = plsc.load_gather(cnts_v, [li + 16, zeros])
  off = jnp.sum(jnp.where(li < w, c0, 0)) + jnp.sum(
      jnp.where(li + 16 < w, c1, 0))
  n_total = jnp.sum(c0) + jnp.sum(c1)
  zoff = n_total + w * NCOLS - off

  def step(c, carry):
    off_nz, off_z = carry
    v = xrow[pl.ds(c * L, L)]
    m = v != 0.0
    mi = m.astype(jnp.int32)
    excl = plsc.cumsum(mi) - mi
    dest = jnp.where(m, off_nz + excl, off_z + (li - excl))
    code = jnp.where(m, w * NCOLS + c * L + li, 0)
    g = c >> 3
    o = (c & 7) * L
    dest_b[g, pl.ds(o, L)] = dest
    val_b[g, pl.ds(o, L)] = code
    pc = plsc.all_reduce_population_count(m)
    return off_nz + pc, off_z + (L - pc)

  init = (jnp.full((L,), off, jnp.int32), jnp.full((L,), zoff, jnp.int32))
  lax.fori_loop(0, CHUNKS, step, init)

  copies = [
      pltpu.async_copy(val_b.at[g], codes_hbm.at[dest_b.at[g]], sem)
      for g in range(GROUPS)
  ]
  for cp in copies:
    cp.wait()


def _decode_body(codes_ref, rows_ref, cols_ref):
  c = codes_ref[...]
  rows_ref[...] = c >> 12
  cols_ref[...] = c & (NCOLS - 1)


_decode = pl.pallas_call(
    _decode_body,
    out_shape=(
        jax.ShapeDtypeStruct((NROWS, NCOLS), jnp.int32),
        jax.ShapeDtypeStruct((NROWS, NCOLS), jnp.int32),
    ),
)


def kernel(x):
  counts = _count_kernel(x)
  codes = _scatter_kernel(x, counts)
  rows, cols = _decode(codes.reshape(NROWS, NCOLS))
  return rows.reshape(-1), cols.reshape(-1)


# speedup vs baseline: 13.6362x; 13.6362x over previous
import functools

import jax
import jax.numpy as jnp
from jax import lax
from jax.experimental import pallas as pl
from jax.experimental.pallas import tpu as pltpu
from jax.experimental.pallas import tpu_sc as plsc

NC = 2
NS = 16
L = 16
NROWS = 32
NCOLS = 4096
N = NROWS * NCOLS
CHUNKS = NCOLS // L
GROUPS = NCOLS // 128

_mesh = plsc.VectorSubcoreMesh(
    core_axis_name="c", subcore_axis_name="s", num_cores=NC, num_subcores=NS
)

_params = pltpu.CompilerParams(needs_layout_passes=False)


def _worker_id():
  return lax.axis_index("s") * NC + lax.axis_index("c")


@functools.partial(
    pl.kernel,
    compiler_params=_params,
    out_type=jax.ShapeDtypeStruct((NROWS, L), jnp.int32),
    mesh=_mesh,
    scratch_types=[
        pltpu.VMEM((NCOLS,), jnp.float32),
        pltpu.VMEM((L,), jnp.int32),
    ],
)
def _count_kernel(x_hbm, counts_hbm, xrow, cnt_v):
  w = _worker_id()
  pltpu.sync_copy(x_hbm.at[w], xrow)

  def step(c, acc):
    v = xrow[pl.ds(c * L, L)]
    return acc + jnp.where(v != 0.0, 1, 0).astype(jnp.int32)

  acc = lax.fori_loop(0, CHUNKS, step, jnp.zeros((L,), jnp.int32),
                      unroll=8)
  total = jnp.sum(acc)
  cnt_v[...] = jnp.full((L,), total, jnp.int32)
  pltpu.sync_copy(cnt_v, counts_hbm.at[w])


@functools.partial(
    pl.kernel,
    compiler_params=_params,
    out_type=(
        jax.ShapeDtypeStruct((N,), jnp.int32),
        jax.ShapeDtypeStruct((N,), jnp.int32),
    ),
    mesh=_mesh,
    scratch_types=[
        pltpu.VMEM((NCOLS,), jnp.float32),
        pltpu.VMEM((NROWS, L), jnp.int32),
        pltpu.VMEM((GROUPS, 128), jnp.int32),
        pltpu.VMEM((NCOLS,), jnp.int32),
        pltpu.VMEM((NCOLS,), jnp.int32),
        pltpu.SemaphoreType.DMA,
    ],
)
def _scatter_kernel(x_hbm, counts_hbm, rows_hbm, cols_hbm, xrow, cnts_v,
                    dest_b, rval_b, cval_b, sem):
  w = _worker_id()
  pltpu.sync_copy(counts_hbm, cnts_v)

  li = jnp.arange(L, dtype=jnp.int32)
  zeros = jnp.zeros((L,), jnp.int32)
  c0 = plsc.load_gather(cnts_v, [li, zeros])
  c1 = plsc.load_gather(cnts_v, [li + 16, zeros])
  off = jnp.sum(jnp.where(li < w, c0, 0)) + jnp.sum(
      jnp.where(li + 16 < w, c1, 0))
  n_w = jnp.sum(jnp.where(li == w, c0, 0)) + jnp.sum(
      jnp.where(li + 16 == w, c1, 0))
  n_total = jnp.sum(c0) + jnp.sum(c1)
  zoff = n_total + w * NCOLS - off

  fast = jnp.logical_and(n_w == NCOLS, off % 8 == 0)

  @pl.when(fast)
  def _fast():
    def fill(c, _):
      colv = c * L + li
      rval_b[pl.ds(c * L, L)] = jnp.full((L,), w, jnp.int32)
      cval_b[pl.ds(c * L, L)] = colv
      return 0

    lax.fori_loop(0, CHUNKS, fill, 0, unroll=8)
    o = pl.multiple_of(off, 8)
    pltpu.sync_copy(rval_b, rows_hbm.at[pl.ds(o, NCOLS)])
    pltpu.sync_copy(cval_b, cols_hbm.at[pl.ds(o, NCOLS)])

  @pl.when(jnp.logical_not(fast))
  def _general():
    pltpu.sync_copy(x_hbm.at[w], xrow)

    def step(c, carry):
      off_nz, off_z = carry
      v = xrow[pl.ds(c * L, L)]
      m = v != 0.0
      mi = m.astype(jnp.int32)
      excl = plsc.cumsum(mi) - mi
      dest = jnp.where(m, off_nz + excl, off_z + (li - excl))
      g = c >> 3
      o = (c & 7) * L
      dest_b[g, pl.ds(o, L)] = dest
      rval_b[pl.ds(c * L, L)] = jnp.where(m, w, 0)
      cval_b[pl.ds(c * L, L)] = jnp.where(m, c * L + li, 0)
      pc = plsc.all_reduce_population_count(m)
      return off_nz + pc, off_z + (L - pc)

    init = (jnp.full((L,), off, jnp.int32), jnp.full((L,), zoff, jnp.int32))
    lax.fori_loop(0, CHUNKS, step, init)

    copies = []
    for g in range(GROUPS):
      copies.append(pltpu.async_copy(
          rval_b.at[pl.ds(g * 128, 128)], rows_hbm.at[dest_b.at[g]], sem))
      copies.append(pltpu.async_copy(
          cval_b.at[pl.ds(g * 128, 128)], cols_hbm.at[dest_b.at[g]], sem))
    for cp in copies:
      cp.wait()


def kernel(x):
  counts = _count_kernel(x)
  rows, cols = _scatter_kernel(x, counts)
  return rows, cols


# speedup vs baseline: 14.9445x; 1.0959x over previous
import functools

import jax
import jax.numpy as jnp
from jax import lax
from jax.experimental import pallas as pl
from jax.experimental.pallas import tpu as pltpu
from jax.experimental.pallas import tpu_sc as plsc

NC = 2
NS = 16
L = 16
NROWS = 32
NCOLS = 4096
N = NROWS * NCOLS
CHUNKS = NCOLS // L
GROUPS = NCOLS // 128

_mesh = plsc.VectorSubcoreMesh(
    core_axis_name="c", subcore_axis_name="s", num_cores=NC, num_subcores=NS
)

_params = pltpu.CompilerParams(needs_layout_passes=False)


@functools.partial(
    pl.kernel,
    compiler_params=_params,
    out_type=(
        jax.ShapeDtypeStruct((N,), jnp.int32),
        jax.ShapeDtypeStruct((N,), jnp.int32),
    ),
    mesh=_mesh,
    scratch_types=[
        pltpu.VMEM((2, NCOLS), jnp.float32),
        pltpu.VMEM((NCOLS,), jnp.float32),
        pltpu.VMEM((NS, L), jnp.int32),
        pltpu.VMEM((L,), jnp.int32),
        pltpu.VMEM((NCOLS,), jnp.int32),
        pltpu.VMEM((NCOLS,), jnp.int32),
        pltpu.VMEM((GROUPS, 128), jnp.int32),
        pltpu.VMEM_SHARED((3 * NS, L), jnp.int32),
        pltpu.SemaphoreType.DMA,
    ],
)
def _nonzero_kernel(x_hbm, rows_hbm, cols_hbm, x2, xrow, cnts_v, pub_v,
                    cval_b, rval_b, dest_b, sh_counts, sem):
  c = lax.axis_index("c")
  s = lax.axis_index("s")
  w = c * NS + s
  li = jnp.arange(L, dtype=jnp.int32)

  pltpu.sync_copy(x_hbm.at[pl.ds(2 * s, 2)], x2)

  def cstep(k, accs):
    a0, a1 = accs
    v0 = x2[0, pl.ds(k * L, L)]
    v1 = x2[1, pl.ds(k * L, L)]
    a0 = a0 + jnp.where(v0 != 0.0, 1, 0).astype(jnp.int32)
    a1 = a1 + jnp.where(v1 != 0.0, 1, 0).astype(jnp.int32)
    return a0, a1

  z16 = jnp.zeros((L,), jnp.int32)
  a0, a1 = lax.fori_loop(0, CHUNKS, cstep, (z16, z16), unroll=8)
  t0 = jnp.sum(a0)
  t1 = jnp.sum(a1)
  pub_v[...] = jnp.where(li == 0, t0, jnp.where(li == 1, t1, 0))
  pltpu.sync_copy(pub_v, sh_counts.at[2 * NS + s])

  plsc.subcore_barrier()

  pltpu.sync_copy(sh_counts.at[pl.ds(2 * NS, NS)], cnts_v)
  c_lo = plsc.load_gather(cnts_v, [li >> 1, li & 1])
  hi = li + NS
  c_hi = plsc.load_gather(cnts_v, [hi >> 1, hi & 1])
  off = jnp.sum(jnp.where(li < w, c_lo, 0)) + jnp.sum(
      jnp.where(hi < w, c_hi, 0))
  n_w = jnp.sum(jnp.where(li == w, c_lo, 0)) + jnp.sum(
      jnp.where(hi == w, c_hi, 0))
  n_total = jnp.sum(c_lo) + jnp.sum(c_hi)
  zoff = n_total + w * NCOLS - off

  fast = jnp.logical_and(n_w == NCOLS, off % 8 == 0)

  @pl.when(fast)
  def _fast():
    def fill(k, _):
      rval_b[pl.ds(k * L, L)] = jnp.full((L,), w, jnp.int32)
      cval_b[pl.ds(k * L, L)] = k * L + li
      return 0

    lax.fori_loop(0, CHUNKS, fill, 0, unroll=8)
    o = pl.multiple_of(off, 8)
    pltpu.sync_copy(rval_b, rows_hbm.at[pl.ds(o, NCOLS)])
    pltpu.sync_copy(cval_b, cols_hbm.at[pl.ds(o, NCOLS)])

  @pl.when(jnp.logical_not(fast))
  def _general():
    pltpu.sync_copy(x_hbm.at[w], xrow)

    def step(k, carry):
      off_nz, off_z = carry
      v = xrow[pl.ds(k * L, L)]
      m = v != 0.0
      mi = m.astype(jnp.int32)
      excl = plsc.cumsum(mi) - mi
      dest = jnp.where(m, off_nz + excl, off_z + (li - excl))
      dest_b[k >> 3, pl.ds((k & 7) * L, L)] = dest
      rval_b[pl.ds(k * L, L)] = jnp.where(m, w, 0)
      cval_b[pl.ds(k * L, L)] = jnp.where(m, k * L + li, 0)
      pc = plsc.all_reduce_population_count(m)
      return off_nz + pc, off_z + (L - pc)

    init = (jnp.full((L,), off, jnp.int32), jnp.full((L,), zoff, jnp.int32))
    lax.fori_loop(0, CHUNKS, step, init)

    copies = []
    for g in range(GROUPS):
      copies.append(pltpu.async_copy(
          rval_b.at[pl.ds(g * 128, 128)], rows_hbm.at[dest_b.at[g]], sem))
      copies.append(pltpu.async_copy(
          cval_b.at[pl.ds(g * 128, 128)], cols_hbm.at[dest_b.at[g]], sem))
    for cp in copies:
      cp.wait()


def kernel(x):
  rows, cols = _nonzero_kernel(x)
  return rows, cols


# speedup vs baseline: 16.3169x; 1.0918x over previous
import functools

import jax
import jax.numpy as jnp
from jax import lax
from jax.experimental import pallas as pl
from jax.experimental.pallas import tpu as pltpu
from jax.experimental.pallas import tpu_sc as plsc

NC = 2
NS = 16
L = 16
NROWS = 32
NCOLS = 4096
N = NROWS * NCOLS
CHUNKS = NCOLS // L
GROUPS = NCOLS // 128

_mesh = plsc.VectorSubcoreMesh(
    core_axis_name="c", subcore_axis_name="s", num_cores=NC, num_subcores=NS
)

_params = pltpu.CompilerParams(needs_layout_passes=False)


@functools.partial(
    pl.kernel,
    compiler_params=_params,
    out_type=(
        jax.ShapeDtypeStruct((N,), jnp.int32),
        jax.ShapeDtypeStruct((N,), jnp.int32),
    ),
    mesh=_mesh,
    scratch_types=[
        pltpu.VMEM((2, NCOLS), jnp.float32),
        pltpu.VMEM((NCOLS,), jnp.float32),
        pltpu.VMEM((NS, L), jnp.int32),
        pltpu.VMEM((L,), jnp.int32),
        pltpu.VMEM((NCOLS,), jnp.int32),
        pltpu.VMEM((NCOLS,), jnp.int32),
        pltpu.VMEM((GROUPS, 128), jnp.int32),
        pltpu.VMEM_SHARED((3 * NS, L), jnp.int32),
        pltpu.SemaphoreType.DMA,
    ],
)
def _nonzero_kernel(x_hbm, rows_hbm, cols_hbm, x2, xrow, cnts_v, pub_v,
                    cval_b, rval_b, dest_b, sh_counts, sem):
  c = lax.axis_index("c")
  s = lax.axis_index("s")
  w = c * NS + s
  li = jnp.arange(L, dtype=jnp.int32)

  pltpu.sync_copy(x_hbm.at[pl.ds(2 * s, 2)], x2)
  wsplat = jnp.full((L,), w, jnp.int32)

  def cstep(k, accs):
    a0, a1 = accs
    v0 = x2[0, pl.ds(k * L, L)]
    v1 = x2[1, pl.ds(k * L, L)]
    a0 = a0 + jnp.where(v0 != 0.0, 1, 0).astype(jnp.int32)
    a1 = a1 + jnp.where(v1 != 0.0, 1, 0).astype(jnp.int32)
    rval_b[pl.ds(k * L, L)] = wsplat
    cval_b[pl.ds(k * L, L)] = k * L + li
    return a0, a1

  z16 = jnp.zeros((L,), jnp.int32)
  a0, a1 = lax.fori_loop(0, CHUNKS, cstep, (z16, z16), unroll=8)
  t0 = jnp.sum(a0)
  t1 = jnp.sum(a1)
  pub_v[...] = jnp.where(li == 0, t0, jnp.where(li == 1, t1, 0))
  pltpu.sync_copy(pub_v, sh_counts.at[2 * NS + s])

  plsc.subcore_barrier()

  pltpu.sync_copy(sh_counts.at[pl.ds(2 * NS, NS)], cnts_v)
  c_lo = plsc.load_gather(cnts_v, [li >> 1, li & 1])
  hi = li + NS
  c_hi = plsc.load_gather(cnts_v, [hi >> 1, hi & 1])
  off = jnp.sum(jnp.where(li < w, c_lo, 0)) + jnp.sum(
      jnp.where(hi < w, c_hi, 0))
  n_w = jnp.sum(jnp.where(li == w, c_lo, 0)) + jnp.sum(
      jnp.where(hi == w, c_hi, 0))
  n_total = jnp.sum(c_lo) + jnp.sum(c_hi)
  zoff = n_total + w * NCOLS - off

  fast = jnp.logical_and(n_w == NCOLS, off % 8 == 0)

  @pl.when(fast)
  def _fast():
    o = pl.multiple_of(off, 8)
    cp_r = pltpu.async_copy(rval_b, rows_hbm.at[pl.ds(o, NCOLS)], sem)
    cp_c = pltpu.async_copy(cval_b, cols_hbm.at[pl.ds(o, NCOLS)], sem)
    cp_r.wait()
    cp_c.wait()

  @pl.when(jnp.logical_not(fast))
  def _general():
    pltpu.sync_copy(x_hbm.at[w], xrow)

    def step(k, carry):
      off_nz, off_z = carry
      v = xrow[pl.ds(k * L, L)]
      m = v != 0.0
      mi = m.astype(jnp.int32)
      excl = plsc.cumsum(mi) - mi
      dest = jnp.where(m, off_nz + excl, off_z + (li - excl))
      dest_b[k >> 3, pl.ds((k & 7) * L, L)] = dest
      rval_b[pl.ds(k * L, L)] = jnp.where(m, w, 0)
      cval_b[pl.ds(k * L, L)] = jnp.where(m, k * L + li, 0)
      pc = plsc.all_reduce_population_count(m)
      return off_nz + pc, off_z + (L - pc)

    init = (jnp.full((L,), off, jnp.int32), jnp.full((L,), zoff, jnp.int32))
    lax.fori_loop(0, CHUNKS, step, init)

    copies = []
    for g in range(GROUPS):
      copies.append(pltpu.async_copy(
          rval_b.at[pl.ds(g * 128, 128)], rows_hbm.at[dest_b.at[g]], sem))
      copies.append(pltpu.async_copy(
          cval_b.at[pl.ds(g * 128, 128)], cols_hbm.at[dest_b.at[g]], sem))
    for cp in copies:
      cp.wait()


def kernel(x):
  rows, cols = _nonzero_kernel(x)
  return rows, cols
